# Initial kernel scaffold; baseline (speedup 1.0000x reference)
import jax
import jax.numpy as jnp
from jax.experimental import pallas as pl


def kernel(input_ids, embeddings, positional_id):
    raise NotImplementedError("write your pallas kernel here")


# speedup vs baseline: 1.2728x; 1.2728x over previous
import functools

import jax
import jax.numpy as jnp
from jax import lax
from jax.experimental import pallas as pl
from jax.experimental.pallas import tpu as pltpu
from jax.experimental.pallas import tpu_sc as plsc

_LANES = 16


@functools.lru_cache(maxsize=None)
def _make_sc_embed(B, D, NC, NS, CH):
    NW = NC * NS
    b_per_w = B // NW
    n_chunks = b_per_w // CH
    mesh = plsc.VectorSubcoreMesh(core_axis_name="c", subcore_axis_name="s")

    @functools.partial(
        pl.kernel,
        mesh=mesh,
        out_type=jax.ShapeDtypeStruct((B, D), jnp.float32),
        scratch_types=[
            pltpu.VMEM((n_chunks, CH), jnp.int32),
            pltpu.VMEM((D,), jnp.float32),
            pltpu.VMEM((CH, D), jnp.float32),
            pltpu.VMEM((CH, D), jnp.float32),
            pltpu.SemaphoreType.DMA,
            pltpu.SemaphoreType.DMA,
            pltpu.SemaphoreType.DMA,
            pltpu.SemaphoreType.DMA,
        ],
    )
    def k(ids_hbm, table_hbm, pos_hbm, out_hbm,
          idx_v, pos_v, buf0, buf1, g0, g1, o0, o1):
        wid = lax.axis_index("s") * NC + lax.axis_index("c")
        base = wid * b_per_w
        pltpu.sync_copy(ids_hbm.at[wid], idx_v)
        pltpu.sync_copy(pos_hbm, pos_v)
        bufs = (buf0, buf1)
        gsem = (g0, g1)
        osem = (o0, o1)

        def add_pos(buf):
            def col_body(v, _):
                sl = pl.ds(v * _LANES, _LANES)
                pv = pos_v[sl]

                def row_body(r, _):
                    buf[r, sl] = buf[r, sl] + pv
                    return 0

                lax.fori_loop(0, CH, row_body, 0, unroll=4)
                return 0

            lax.fori_loop(0, D // _LANES, col_body, 0)

        gcp = [None, None]
        ocp = [None, None]
        gcp[0] = pltpu.async_copy(table_hbm.at[idx_v.at[0]], bufs[0], gsem[0])
        for c in range(n_chunks):
            s = c & 1
            gcp[s].wait()
            nxt = c + 1
            if nxt < n_chunks:
                sp = nxt & 1
                if ocp[sp] is not None:
                    ocp[sp].wait()
                gcp[sp] = pltpu.async_copy(
                    table_hbm.at[idx_v.at[nxt]], bufs[sp], gsem[sp])
            add_pos(bufs[s])
            ocp[s] = pltpu.async_copy(
                bufs[s], out_hbm.at[pl.ds(base + c * CH, CH)], osem[s])
        for s in range(2):
            if ocp[s] is not None:
                ocp[s].wait()

    return k


def kernel(input_ids, embeddings, positional_id):
    B = input_ids.shape[0]
    D = embeddings.shape[1]
    info = plsc.get_sparse_core_info()
    NC, NS = info.num_cores, info.num_subcores
    CH = 16
    ids3 = input_ids.astype(jnp.int32).reshape(NC * NS, -1, CH)
    pos_f = positional_id[0, :D].astype(jnp.float32)
    k = _make_sc_embed(B, D, NC, NS, CH)
    return k(ids3, embeddings, pos_f)
